# baseline (speedup 1.0000x reference)
import jax
import jax.numpy as jnp
from jax.experimental import pallas as pl
from jax.experimental.pallas import tpu as pltpu

_TM = 1024
_NQ = 4


def _ffn_kernel(x_ref, w1_hbm, b1_ref, w2_hbm, b2_ref, o_ref,
                w1v, w2v, sem1, sem2):
    i = pl.program_id(0)
    d_ff = w2v.shape[0]
    q = d_ff // _NQ

    def w1_copy(c):
        return pltpu.make_async_copy(
            w1_hbm.at[:, pl.ds(c * q, q)],
            w1v.at[:, pl.ds(c * q, q)], sem1.at[c])

    def w2_copy(c):
        return pltpu.make_async_copy(
            w2_hbm.at[pl.ds(c * q, q), :],
            w2v.at[pl.ds(c * q, q), :], sem2.at[c])

    @pl.when(i == 0)
    def _():
        for c in range(_NQ):
            w1_copy(c).start()
            w2_copy(c).start()

    for c in range(_NQ):
        @pl.when(i == 0)
        def _(c=c):
            w1_copy(c).wait()
            w2_copy(c).wait()

        hq = jnp.dot(x_ref[...], w1v[:, pl.ds(c * q, q)],
                     preferred_element_type=jnp.float32)
        hq = jnp.maximum(hq + b1_ref[:, pl.ds(c * q, q)], 0.0)
        p = jnp.dot(hq, w2v[pl.ds(c * q, q), :],
                    preferred_element_type=jnp.float32)
        if c == 0:
            o_ref[...] = p + b2_ref[...]
        else:
            o_ref[...] += p


def kernel(x, w1, b1, w2, b2):
    B, S, d_model = x.shape
    d_ff = w1.shape[1]
    M = B * S

    x2d = x.reshape(M, d_model)
    b1_2d = b1.reshape(1, d_ff)
    b2_2d = b2.reshape(1, d_model)

    out2d = pl.pallas_call(
        _ffn_kernel,
        out_shape=jax.ShapeDtypeStruct((M, d_model), jnp.float32),
        grid=(M // _TM,),
        in_specs=[
            pl.BlockSpec((_TM, d_model), lambda i: (i, 0)),
            pl.BlockSpec(memory_space=pltpu.MemorySpace.HBM),
            pl.BlockSpec((1, d_ff), lambda i: (0, 0)),
            pl.BlockSpec(memory_space=pltpu.MemorySpace.HBM),
            pl.BlockSpec((1, d_model), lambda i: (0, 0)),
        ],
        out_specs=pl.BlockSpec((_TM, d_model), lambda i: (i, 0)),
        scratch_shapes=[
            pltpu.VMEM((d_model, d_ff), jnp.float32),
            pltpu.VMEM((d_ff, d_model), jnp.float32),
            pltpu.SemaphoreType.DMA((_NQ,)),
            pltpu.SemaphoreType.DMA((_NQ,)),
        ],
        compiler_params=pltpu.CompilerParams(
            dimension_semantics=("arbitrary",),
            vmem_limit_bytes=60 * 1024 * 1024,
        ),
        cost_estimate=pl.CostEstimate(
            flops=4 * M * d_model * d_ff,
            transcendentals=0,
            bytes_accessed=(x2d.size + w1.size + b1.size + w2.size + b2.size
                            + M * d_model) * 4,
        ),
    )(x2d, w1, b1_2d, w2, b2_2d)

    return out2d.reshape(B, S, d_model)


# speedup vs baseline: 1.0100x; 1.0100x over previous
import jax
import jax.numpy as jnp
from jax.experimental import pallas as pl
from jax.experimental.pallas import tpu as pltpu

_TM = 512
_NQ = 4


def _ffn_kernel(x_ref, w1_hbm, b1_ref, w2_hbm, b2_ref, o_ref,
                w1v, w2v, sem1, sem2):
    i = pl.program_id(0)
    d_model = w1v.shape[0]
    d_ff = w2v.shape[0]
    q1 = d_model // _NQ
    q2 = d_ff // _NQ

    def w1_copy(q):
        return pltpu.make_async_copy(
            w1_hbm.at[pl.ds(q * q1, q1), :],
            w1v.at[pl.ds(q * q1, q1), :], sem1.at[q])

    def w2_copy(q):
        return pltpu.make_async_copy(
            w2_hbm.at[pl.ds(q * q2, q2), :],
            w2v.at[pl.ds(q * q2, q2), :], sem2.at[q])

    @pl.when(i == 0)
    def _first():
        for q in range(_NQ):
            w1_copy(q).start()
        for q in range(_NQ):
            w2_copy(q).start()
        x_val = x_ref[...]
        h = None
        for q in range(_NQ):
            w1_copy(q).wait()
            p = jnp.dot(x_val[:, q * q1:(q + 1) * q1],
                        w1v[pl.ds(q * q1, q1), :],
                        preferred_element_type=jnp.float32)
            h = p if h is None else h + p
        h = jnp.maximum(h + b1_ref[...], 0.0)
        out = None
        for q in range(_NQ):
            w2_copy(q).wait()
            p = jnp.dot(h[:, q * q2:(q + 1) * q2],
                        w2v[pl.ds(q * q2, q2), :],
                        preferred_element_type=jnp.float32)
            out = p if out is None else out + p
        o_ref[...] = out + b2_ref[...]

    @pl.when(i > 0)
    def _rest():
        h = jnp.dot(x_ref[...], w1v[...], preferred_element_type=jnp.float32)
        h = jnp.maximum(h + b1_ref[...], 0.0)
        out = jnp.dot(h, w2v[...], preferred_element_type=jnp.float32)
        o_ref[...] = out + b2_ref[...]


def kernel(x, w1, b1, w2, b2):
    B, S, d_model = x.shape
    d_ff = w1.shape[1]
    M = B * S

    x2d = x.reshape(M, d_model)
    b1_2d = b1.reshape(1, d_ff)
    b2_2d = b2.reshape(1, d_model)

    out2d = pl.pallas_call(
        _ffn_kernel,
        out_shape=jax.ShapeDtypeStruct((M, d_model), jnp.float32),
        grid=(M // _TM,),
        in_specs=[
            pl.BlockSpec((_TM, d_model), lambda i: (i, 0)),
            pl.BlockSpec(memory_space=pltpu.MemorySpace.HBM),
            pl.BlockSpec((1, d_ff), lambda i: (0, 0)),
            pl.BlockSpec(memory_space=pltpu.MemorySpace.HBM),
            pl.BlockSpec((1, d_model), lambda i: (0, 0)),
        ],
        out_specs=pl.BlockSpec((_TM, d_model), lambda i: (i, 0)),
        scratch_shapes=[
            pltpu.VMEM((d_model, d_ff), jnp.float32),
            pltpu.VMEM((d_ff, d_model), jnp.float32),
            pltpu.SemaphoreType.DMA((_NQ,)),
            pltpu.SemaphoreType.DMA((_NQ,)),
        ],
        compiler_params=pltpu.CompilerParams(
            dimension_semantics=("arbitrary",),
            vmem_limit_bytes=60 * 1024 * 1024,
        ),
        cost_estimate=pl.CostEstimate(
            flops=4 * M * d_model * d_ff,
            transcendentals=0,
            bytes_accessed=(x2d.size + w1.size + b1.size + w2.size + b2.size
                            + M * d_model) * 4,
        ),
    )(x2d, w1, b1_2d, w2, b2_2d)

    return out2d.reshape(B, S, d_model)
